# baseline (speedup 1.0000x reference)
import functools

import jax
import jax.numpy as jnp
import numpy as np
from jax.experimental import pallas as pl
from jax.experimental.pallas import tpu as pltpu

B = 128
A = 100000
BC = 512
K = (A + BC - 1) // BC

LOGP = np.float32(np.log(np.float64(np.float32(1.0 / A))))

_KS1 = np.uint32(42)
_KS2 = np.uint32(42 ^ 0x1BD11BDA)
_ROT_A = (13, 15, 26, 6)
_ROT_B = (17, 29, 16, 24)


def _rounds(x0, x1, rots):
    for d in rots:
        x0 = x0 + x1
        x1 = ((x1 << np.uint32(d)) | (x1 >> np.uint32(32 - d))) ^ x0
    return x0, x1


def _threefry_bits(i):
    x0 = jnp.zeros_like(i)
    x1 = i + _KS1
    x0, x1 = _rounds(x0, x1, _ROT_A)
    x0, x1 = x0 + _KS1, x1 + _KS2 + np.uint32(1)
    x0, x1 = _rounds(x0, x1, _ROT_B)
    x0, x1 = x0 + _KS2, x1 + np.uint32(0 + 2)
    x0, x1 = _rounds(x0, x1, _ROT_A)
    x0, x1 = x0 + np.uint32(0), x1 + _KS1 + np.uint32(3)
    x0, x1 = _rounds(x0, x1, _ROT_B)
    x0, x1 = x0 + _KS1, x1 + _KS2 + np.uint32(4)
    x0, x1 = _rounds(x0, x1, _ROT_A)
    x0, x1 = x0 + _KS2, x1 + np.uint32(0 + 5)
    return x0 ^ x1


def _walk_kernel(actions_ref, logprob_ref, bv_ref, bi_ref):
    k = pl.program_id(0)

    logprob_ref[...] = jnp.full((B, BC), LOGP, dtype=jnp.float32)

    row = jax.lax.broadcasted_iota(jnp.int32, (B, BC), 0)
    col = jax.lax.broadcasted_iota(jnp.int32, (B, BC), 1) + k * BC
    lin = (row * A + col).astype(jnp.uint32)
    bits = _threefry_bits(lin)
    m = (bits >> np.uint32(9)).astype(jnp.int32)
    m = jnp.where(col < A, m, -1)

    bmax = jnp.max(m, axis=1, keepdims=True)
    cand = jnp.where(m == bmax, col, jnp.int32(2**31 - 1))
    bidx = jnp.min(cand, axis=1, keepdims=True)

    @pl.when(k == 0)
    def _init():
        bv_ref[...] = bmax
        bi_ref[...] = bidx

    @pl.when(k > 0)
    def _combine():
        better = bmax > bv_ref[...]
        bi_ref[...] = jnp.where(better, bidx, bi_ref[...])
        bv_ref[...] = jnp.maximum(bmax, bv_ref[...])

    @pl.when(k == K - 1)
    def _emit():
        actions_ref[...] = bi_ref[...]


@functools.partial(jax.jit, static_argnames=())
def _run():
    actions2d, logprob = pl.pallas_call(
        _walk_kernel,
        grid=(K,),
        out_specs=[
            pl.BlockSpec((B, 1), lambda k: (0, 0)),
            pl.BlockSpec((B, BC), lambda k: (0, k)),
        ],
        out_shape=[
            jax.ShapeDtypeStruct((B, 1), jnp.int32),
            jax.ShapeDtypeStruct((B, A), jnp.float32),
        ],
        scratch_shapes=[
            pltpu.VMEM((B, 1), jnp.int32),
            pltpu.VMEM((B, 1), jnp.int32),
        ],
    )()
    return actions2d.reshape(B), logprob


def kernel(state):
    del state
    return _run()
